# baseline (speedup 1.0000x reference)
import jax
import jax.numpy as jnp
from jax import lax
from jax.experimental import pallas as pl
from jax.experimental.pallas import tpu as pltpu
from jax.experimental.pallas import tpu_sc as plsc

B = 16384
D = 16
NC = 2
NS = 16
NW = NC * NS
BPW = B // NW
G = BPW // D

V = 1_000_000
VP = 1_000_064
LIN = 16 * VP
W = 4096
NPW = 15
TS = V - 64


def _fused_body(users_hbm, items_hbm, vu_hbm, vi_hbm, ub_hbm, ib_hbm,
                tu_hbm, ti_hbm, out_hbm, lu_hbm, li_hbm,
                dbuf0, dbuf1, idx_u, idx_i, bias_u, bias_i,
                out_v, tail_u, tail_i, *rest):
    cols_u = rest[:D]
    cols_i = rest[D:2 * D]
    s0, s1, so0, so1, gdsem, gsem = rest[2 * D:]
    cid = lax.axis_index("c")
    sid = lax.axis_index("s")
    wid = sid * NC + cid
    tbl = wid // 16
    o = wid % 16
    base = wid * BPW
    bufs = (dbuf0, dbuf1)
    sin = (s0, s1)
    sout = (so0, so1)

    def xbarrier():
        plsc.subcore_barrier()

        @pl.when(sid == 0)
        def _():
            pltpu.semaphore_signal(
                gsem, 1, device_id={"c": 1 - cid, "s": 0},
                device_id_type=pl.DeviceIdType.MESH)
            pl.semaphore_wait(gsem, 1)

        plsc.subcore_barrier()

    pltpu.sync_copy(users_hbm.at[pl.ds(base, BPW)], idx_u)
    pltpu.sync_copy(items_hbm.at[pl.ds(base, BPW)], idx_i)
    pltpu.sync_copy(tu_hbm, tail_u)
    pltpu.sync_copy(ti_hbm, tail_i)

    def detile_half(src, dst, tr):
        def chunk_src(i):
            return src.at[tr, :, pl.ds((o * NPW + i) * W, W)]

        in_flight = [None, None]
        out_flight = [[], []]
        in_flight[0] = pltpu.async_copy(chunk_src(0), bufs[0], sin[0])
        for i in range(NPW):
            b = i % 2
            in_flight[b].wait()
            nxt = i + 1
            if nxt < NPW:
                nb = nxt % 2
                for c in out_flight[nb]:
                    c.wait()
                in_flight[nb] = pltpu.async_copy(chunk_src(nxt), bufs[nb], sin[nb])
            c0 = (o * NPW + i) * W
            out_flight[b] = [
                pltpu.async_copy(bufs[b].at[fr],
                                 dst.at[pl.ds((tr * 8 + fr) * VP + c0, W)],
                                 sout[b])
                for fr in range(8)
            ]
        for plist in out_flight:
            for c in plist:
                c.wait()

        @pl.when(o < 4)
        def _():
            c0 = (16 * NPW + o) * W
            pltpu.sync_copy(src.at[tr, :, pl.ds(c0, W)], bufs[0])
            for fr in range(8):
                pltpu.sync_copy(bufs[0].at[fr],
                                dst.at[pl.ds((tr * 8 + fr) * VP + c0, W)])

        @pl.when(o == 15)
        def _():
            c0 = (16 * NPW + 4) * W
            pltpu.sync_copy(src.at[tr, :, pl.ds(c0, 512)],
                            bufs[0].at[:, pl.ds(0, 512)])
            for fr in range(8):
                pltpu.sync_copy(bufs[0].at[fr, pl.ds(0, 512)],
                                dst.at[pl.ds((tr * 8 + fr) * VP + c0, 512)])

    def detile_phase(tr):
        @pl.when(tbl == 0)
        def _():
            detile_half(vu_hbm, lu_hbm, tr)

        @pl.when(tbl == 1)
        def _():
            detile_half(vi_hbm, li_hbm, tr)

    detile_phase(0)
    xbarrier()

    copies = [
        pltpu.async_copy(ub_hbm.at[idx_u], bias_u, gdsem),
        pltpu.async_copy(ib_hbm.at[idx_i], bias_i, gdsem),
    ]
    for f in range(8):
        copies.append(pltpu.async_copy(
            lu_hbm.at[pl.ds(f * VP, V)].at[idx_u], cols_u[f], gdsem))
        copies.append(pltpu.async_copy(
            li_hbm.at[pl.ds(f * VP, V)].at[idx_i], cols_i[f], gdsem))
    detile_phase(1)
    xbarrier()

    for f in range(8, D):
        copies.append(pltpu.async_copy(
            lu_hbm.at[pl.ds(f * VP, V)].at[idx_u], cols_u[f], gdsem))
        copies.append(pltpu.async_copy(
            li_hbm.at[pl.ds(f * VP, V)].at[idx_i], cols_i[f], gdsem))
    for c in copies:
        c.wait()

    @pl.loop(0, G)
    def _(g):
        r0 = g * D
        iu = idx_u[pl.ds(r0, D)]
        ii = idx_i[pl.ds(r0, D)]
        mu = iu >= TS
        mi = ii >= TS
        tix_u = jnp.maximum(iu - TS, 0)
        tix_i = jnp.maximum(ii - TS, 0)
        acc = bias_u[pl.ds(r0, D)] + bias_i[pl.ds(r0, D)]
        for f in range(D):
            uf = cols_u[f][pl.ds(r0, D)]
            vf = cols_i[f][pl.ds(r0, D)]
            uf = jnp.where(mu, plsc.load_gather(tail_u, [tix_u + f * 64]), uf)
            vf = jnp.where(mi, plsc.load_gather(tail_i, [tix_i + f * 64]), vf)
            acc = acc + uf * vf
        out_v[pl.ds(r0, D)] = acc

    pltpu.sync_copy(out_v, out_hbm.at[pl.ds(base, BPW)])


def kernel(users, items, user_embeddings, item_embeddings, user_biases, item_biases):
    users = users.astype(jnp.int32)
    items = items.astype(jnp.int32)
    vu = user_embeddings.T.reshape(NC, 8, V)
    vi = item_embeddings.T.reshape(NC, 8, V)
    ub = user_biases.reshape(-1)
    ib = item_biases.reshape(-1)
    tu = lax.slice(user_embeddings, (TS, 0), (V, D)).T.reshape(-1)
    ti = lax.slice(item_embeddings, (TS, 0), (V, D)).T.reshape(-1)

    mesh = plsc.VectorSubcoreMesh(core_axis_name="c", subcore_axis_name="s")
    run = pl.kernel(
        _fused_body,
        out_type=(
            jax.ShapeDtypeStruct((B,), jnp.float32),
            jax.ShapeDtypeStruct((LIN,), jnp.float32),
            jax.ShapeDtypeStruct((LIN,), jnp.float32),
        ),
        mesh=mesh,
        scratch_types=[
            pltpu.VMEM((8, W), jnp.float32),
            pltpu.VMEM((8, W), jnp.float32),
            pltpu.VMEM((BPW,), jnp.int32),
            pltpu.VMEM((BPW,), jnp.int32),
            pltpu.VMEM((BPW,), jnp.float32),
            pltpu.VMEM((BPW,), jnp.float32),
            pltpu.VMEM((BPW,), jnp.float32),
            pltpu.VMEM((D * 64,), jnp.float32),
            pltpu.VMEM((D * 64,), jnp.float32),
        ] + [pltpu.VMEM((BPW,), jnp.float32)] * (2 * D) + [
            pltpu.SemaphoreType.DMA,
            pltpu.SemaphoreType.DMA,
            pltpu.SemaphoreType.DMA,
            pltpu.SemaphoreType.DMA,
            pltpu.SemaphoreType.DMA,
            pltpu.SemaphoreType.REGULAR,
        ],
        compiler_params=pltpu.CompilerParams(
            needs_layout_passes=False, use_tc_tiling_on_sc=True),
    )
    out, _, _ = run(users, items, vu, vi, ub, ib, tu, ti)
    return out


# speedup vs baseline: 1.5625x; 1.5625x over previous
import jax
import jax.numpy as jnp
from jax import lax
from jax.experimental import pallas as pl
from jax.experimental.pallas import tpu as pltpu
from jax.experimental.pallas import tpu_sc as plsc

B = 16384
D = 16
NC = 2
NS = 16
NW = NC * NS
BPW = B // NW
G = BPW // D

V = 1_000_000
VP = 1_000_064
LIN = 16 * VP
W = 7808
NPW = 16
NBUF = 2


def _detile_body(vu_hbm, vi_hbm, lu_hbm, li_hbm,
                 buf0, buf1, s0, s1, so0, so1):
    wid = lax.axis_index("s") * NC + lax.axis_index("c")
    tbl = wid // 16
    rem = wid % 16
    tr = rem // 8
    o = rem % 8
    bufs = (buf0, buf1)
    sin = (s0, s1)
    sout = (so0, so1)

    def run(src, dst):
        def chunk_src(i):
            return src.at[tr, :, pl.ds((o * NPW + i) * W, W)]

        in_flight = [None] * NBUF
        out_flight = [[] for _ in range(NBUF)]
        for p in range(NBUF - 1):
            in_flight[p] = pltpu.async_copy(chunk_src(p), bufs[p], sin[p])
        for i in range(NPW):
            b = i % NBUF
            in_flight[b].wait()
            nxt = i + NBUF - 1
            if nxt < NPW:
                nb = nxt % NBUF
                for c in out_flight[nb]:
                    c.wait()
                in_flight[nb] = pltpu.async_copy(chunk_src(nxt), bufs[nb], sin[nb])
            c0 = (o * NPW + i) * W
            out_flight[b] = [
                pltpu.async_copy(bufs[b].at[fr],
                                 dst.at[pl.ds((tr * 8 + fr) * VP + c0, W)],
                                 sout[b])
                for fr in range(8)
            ]
        for plist in out_flight:
            for c in plist:
                c.wait()

        @pl.when(o == 7)
        def _():
            c0 = 8 * NPW * W
            pltpu.sync_copy(src.at[tr, :, pl.ds(c0, 512)],
                            bufs[0].at[:, pl.ds(0, 512)])
            for fr in range(8):
                pltpu.sync_copy(bufs[0].at[fr, pl.ds(0, 512)],
                                dst.at[pl.ds((tr * 8 + fr) * VP + c0, 512)])

    @pl.when(tbl == 0)
    def _():
        run(vu_hbm, lu_hbm)

    @pl.when(tbl == 1)
    def _():
        run(vi_hbm, li_hbm)


TS = V - 64


def _gather_body(users_hbm, items_hbm, lu_hbm, li_hbm, ub_hbm, ib_hbm,
                 tu_hbm, ti_hbm, out_hbm,
                 idx_u, idx_i, cols_u, cols_i, bias_u, bias_i, out_v,
                 tail_u, tail_i, sem):
    wid = lax.axis_index("s") * NC + lax.axis_index("c")
    base = wid * BPW

    pltpu.sync_copy(users_hbm.at[pl.ds(base, BPW)], idx_u)
    pltpu.sync_copy(items_hbm.at[pl.ds(base, BPW)], idx_i)
    pltpu.sync_copy(tu_hbm, tail_u)
    pltpu.sync_copy(ti_hbm, tail_i)

    copies = [
        pltpu.async_copy(ub_hbm.at[idx_u], bias_u, sem),
        pltpu.async_copy(ib_hbm.at[idx_i], bias_i, sem),
    ]
    for f in range(D):
        copies.append(pltpu.async_copy(
            lu_hbm.at[pl.ds(f * VP, V)].at[idx_u], cols_u.at[f], sem))
        copies.append(pltpu.async_copy(
            li_hbm.at[pl.ds(f * VP, V)].at[idx_i], cols_i.at[f], sem))
    for c in copies:
        c.wait()

    @pl.loop(0, G)
    def _(g):
        r0 = g * D
        iu = idx_u[pl.ds(r0, D)]
        ii = idx_i[pl.ds(r0, D)]
        mu = iu >= TS
        mi = ii >= TS
        tix_u = jnp.maximum(iu - TS, 0)
        tix_i = jnp.maximum(ii - TS, 0)
        acc = bias_u[pl.ds(r0, D)] + bias_i[pl.ds(r0, D)]
        for f in range(D):
            uf = cols_u[f, pl.ds(r0, D)]
            vf = cols_i[f, pl.ds(r0, D)]
            uf = jnp.where(mu, plsc.load_gather(tail_u, [tix_u + f * 64]), uf)
            vf = jnp.where(mi, plsc.load_gather(tail_i, [tix_i + f * 64]), vf)
            acc = acc + uf * vf
        out_v[pl.ds(r0, D)] = acc

    pltpu.sync_copy(out_v, out_hbm.at[pl.ds(base, BPW)])


def kernel(users, items, user_embeddings, item_embeddings, user_biases, item_biases):
    users = users.astype(jnp.int32)
    items = items.astype(jnp.int32)
    vu = user_embeddings.T.reshape(NC, 8, V)
    vi = item_embeddings.T.reshape(NC, 8, V)
    ub = user_biases.reshape(-1)
    ib = item_biases.reshape(-1)
    tu = lax.slice(user_embeddings, (TS, 0), (V, D)).T.reshape(-1)
    ti = lax.slice(item_embeddings, (TS, 0), (V, D)).T.reshape(-1)

    mesh = plsc.VectorSubcoreMesh(core_axis_name="c", subcore_axis_name="s")

    detile = pl.kernel(
        _detile_body,
        out_type=(
            jax.ShapeDtypeStruct((LIN,), jnp.float32),
            jax.ShapeDtypeStruct((LIN,), jnp.float32),
        ),
        mesh=mesh,
        scratch_types=[
            pltpu.VMEM((8, W), jnp.float32),
            pltpu.VMEM((8, W), jnp.float32),
            pltpu.SemaphoreType.DMA,
            pltpu.SemaphoreType.DMA,
            pltpu.SemaphoreType.DMA,
            pltpu.SemaphoreType.DMA,
        ],
        compiler_params=pltpu.CompilerParams(
            needs_layout_passes=False, use_tc_tiling_on_sc=True),
    )
    lu, li = detile(vu, vi)

    gather = pl.kernel(
        _gather_body,
        out_type=jax.ShapeDtypeStruct((B,), jnp.float32),
        mesh=mesh,
        scratch_types=[
            pltpu.VMEM((BPW,), jnp.int32),
            pltpu.VMEM((BPW,), jnp.int32),
            pltpu.VMEM((D, BPW), jnp.float32),
            pltpu.VMEM((D, BPW), jnp.float32),
            pltpu.VMEM((BPW,), jnp.float32),
            pltpu.VMEM((BPW,), jnp.float32),
            pltpu.VMEM((BPW,), jnp.float32),
            pltpu.VMEM((D * 64,), jnp.float32),
            pltpu.VMEM((D * 64,), jnp.float32),
            pltpu.SemaphoreType.DMA,
        ],
        compiler_params=pltpu.CompilerParams(
            needs_layout_passes=False, use_tc_tiling_on_sc=False),
    )
    return gather(users, items, lu, li, ub, ib, tu, ti)
